# baseline (speedup 1.0000x reference)
import math

import jax
import jax.numpy as jnp
from jax.experimental import pallas as pl
from jax.experimental.pallas import tpu as pltpu

N = 2048
HEADS = 16
NUM_BUCKETS = 32
MAX_DISTANCE = 128
WW = 4480
VW = 4096
NT = N // 256


def _build(tab_ref, w_ref, v3_ref, hh, r):
    s = jax.lax.broadcasted_iota(jnp.int32, (8, WW), 0)
    z = jax.lax.broadcasted_iota(jnp.int32, (8, WW), 1)
    d = jnp.clip(z - s - (249 + N - 1), -(N - 1), N - 1)
    nb = NUM_BUCKETS // 2
    neg = -d
    ret = jnp.where(neg < 0, nb, 0)
    an = jnp.abs(neg)
    max_exact = nb // 2
    nf = jnp.maximum(an.astype(jnp.float32), 1.0)
    val_large = max_exact + (
        jnp.log(nf / max_exact) / math.log(MAX_DISTANCE / max_exact) * (nb - max_exact)
    ).astype(jnp.int32)
    val_large = jnp.minimum(val_large, nb - 1)
    bucket = ret + jnp.where(an < max_exact, an, val_large)
    acc = jnp.zeros((8, WW), jnp.float32)
    for b in range(NUM_BUCKETS):
        acc = jnp.where(bucket == b, tab_ref[hh, b], acc)
    w_ref[:, :] = acc
    for k in range(32):
        v3_ref[r, 8 * k : 8 * k + 8, :] = w_ref[:, 248 - 8 * k : 248 - 8 * k + VW]


def _block_copy(o_ref, v3_ref, sem_ref, h, r, t):
    src = v3_ref.at[r, :, pl.ds((NT - t) * 256, N)]
    dst = o_ref.at[h, pl.ds(256 * t, 256), :]
    return pltpu.make_async_copy(src, dst, sem_ref.at[r])


def _body(tab_ref, o_ref, w_ref, v3_ref, sem_ref):
    h = pl.program_id(0)
    r = jax.lax.rem(h, 3)
    rn = jax.lax.rem(h + 1, 3)

    @pl.when(h == 0)
    def _prologue():
        _build(tab_ref, w_ref, v3_ref, 0, 0)

    for t in range(NT):
        _block_copy(o_ref, v3_ref, sem_ref, h, r, t).start()

    @pl.when(h >= 2)
    def _reclaim():
        for t in range(NT):
            _block_copy(o_ref, v3_ref, sem_ref, h - 2, rn, t).wait()

    @pl.when(h < HEADS - 1)
    def _build_next():
        _build(tab_ref, w_ref, v3_ref, h + 1, rn)

    @pl.when(h == HEADS - 1)
    def _drain():
        for t in range(NT):
            _block_copy(o_ref, v3_ref, sem_ref, h - 1, jax.lax.rem(h - 1, 3), t).wait()
        for t in range(NT):
            _block_copy(o_ref, v3_ref, sem_ref, h, r, t).wait()


def kernel(n, relative_attention_bias):
    del n
    tab_t = relative_attention_bias.T
    out = pl.pallas_call(
        _body,
        grid=(HEADS,),
        in_specs=[pl.BlockSpec(memory_space=pltpu.SMEM)],
        out_specs=pl.BlockSpec(memory_space=pl.ANY),
        out_shape=jax.ShapeDtypeStruct((HEADS, N, N), jnp.float32),
        scratch_shapes=[
            pltpu.VMEM((8, WW), jnp.float32),
            pltpu.VMEM((3, 256, VW), jnp.float32),
            pltpu.SemaphoreType.DMA((3,)),
        ],
    )(tab_t)
    return out
